# baseline (speedup 1.0000x reference)
import functools

import jax
import jax.numpy as jnp
from jax import lax
from jax.experimental import pallas as pl
from jax.experimental.pallas import tpu as pltpu
from jax.experimental.pallas import tpu_sc as plsc

NUM_EMBEDDINGS = 1000000
FEATURES = 64
BATCH = 16384
FIELDS = 26

NC = 2
NS = 16
NW = NC * NS

B_TOTAL = BATCH * FIELDS
BPW = B_TOTAL // NW
CHUNK = 128
NCHUNK = BPW // CHUNK

_mesh = plsc.VectorSubcoreMesh(core_axis_name="c", subcore_axis_name="s")


@functools.partial(
    pl.kernel,
    mesh=_mesh,
    out_type=jax.ShapeDtypeStruct((B_TOTAL, FEATURES), jnp.float32),
    scratch_types=[
        pltpu.VMEM((NCHUNK, CHUNK), jnp.int32),
        pltpu.VMEM((2, CHUNK, FEATURES), jnp.float32),
        pltpu.SemaphoreType.DMA,
    ],
    compiler_params=pltpu.CompilerParams(use_tc_tiling_on_sc=False),
)
def _embed_gather(idx_hbm, table_hbm, out_hbm, idx_v, rows_v, gsem):
    wid = lax.axis_index("s") * NC + lax.axis_index("c")
    base = wid * BPW

    pltpu.sync_copy(idx_hbm.at[wid], idx_v)

    pltpu.async_copy(table_hbm.at[idx_v.at[0]], rows_v.at[0], gsem)

    def body(g, carry):
        cur = lax.rem(g, 2)
        nxt = lax.rem(g + 1, 2)

        @pl.when(g + 1 < NCHUNK)
        def _():
            pltpu.async_copy(table_hbm.at[idx_v.at[g + 1]], rows_v.at[nxt], gsem)

        pltpu.make_async_copy(
            table_hbm.at[idx_v.at[g]], rows_v.at[cur], gsem
        ).wait()
        pltpu.sync_copy(
            rows_v.at[cur], out_hbm.at[pl.ds(base + g * CHUNK, CHUNK)]
        )
        return carry

    lax.fori_loop(0, NCHUNK, body, 0)


def kernel(inputs, embedding):
    idx = inputs.astype(jnp.int32).reshape(NW, NCHUNK, CHUNK)
    table = jnp.asarray(embedding, jnp.float32)
    out = _embed_gather(idx, table)
    return out.reshape(BATCH, FIELDS, FEATURES)


# speedup vs baseline: 1.0154x; 1.0154x over previous
import functools

import jax
import jax.numpy as jnp
from jax import lax
from jax.experimental import pallas as pl
from jax.experimental.pallas import tpu as pltpu
from jax.experimental.pallas import tpu_sc as plsc

NUM_EMBEDDINGS = 1000000
FEATURES = 64
BATCH = 16384
FIELDS = 26

NC = 2
NS = 16
NW = NC * NS

B_TOTAL = BATCH * FIELDS
BPW = B_TOTAL // NW
CHUNK = 128
NCHUNK = BPW // CHUNK
K = 4
NGROUP = NCHUNK // K
NBUF = 3 * K

_mesh = plsc.VectorSubcoreMesh(core_axis_name="c", subcore_axis_name="s")


@functools.partial(
    pl.kernel,
    mesh=_mesh,
    out_type=jax.ShapeDtypeStruct((B_TOTAL, FEATURES), jnp.float32),
    scratch_types=[
        pltpu.VMEM((NCHUNK, CHUNK), jnp.int32),
        pltpu.VMEM((NBUF, CHUNK, FEATURES), jnp.float32),
        pltpu.SemaphoreType.DMA,
        pltpu.SemaphoreType.DMA,
    ],
    compiler_params=pltpu.CompilerParams(use_tc_tiling_on_sc=False),
)
def _embed_gather(idx_hbm, table_hbm, out_hbm, idx_v, rows_v, gsem, ssem):
    wid = lax.axis_index("s") * NC + lax.axis_index("c")
    base = wid * BPW

    pltpu.sync_copy(idx_hbm.at[wid], idx_v)

    for j in range(K):
        pltpu.async_copy(table_hbm.at[idx_v.at[j]], rows_v.at[j], gsem)

    def body(i, carry):
        @pl.when(i >= 2)
        def _():
            for j in range(K):
                pltpu.make_async_copy(
                    rows_v.at[j], out_hbm.at[pl.ds(base, CHUNK)], ssem
                ).wait()

        @pl.when(i + 1 < NGROUP)
        def _():
            boff = lax.rem(i + 1, 3) * K
            for j in range(K):
                pltpu.async_copy(
                    table_hbm.at[idx_v.at[(i + 1) * K + j]],
                    rows_v.at[boff + j],
                    gsem,
                )

        boff = lax.rem(i, 3) * K
        for j in range(K):
            c = i * K + j
            pltpu.make_async_copy(
                table_hbm.at[idx_v.at[c]], rows_v.at[boff + j], gsem
            ).wait()
            pltpu.async_copy(
                rows_v.at[boff + j],
                out_hbm.at[pl.ds(base + c * CHUNK, CHUNK)],
                ssem,
            )
        return carry

    lax.fori_loop(0, NGROUP, body, 0)

    for j in range(2 * K):
        pltpu.make_async_copy(
            rows_v.at[0], out_hbm.at[pl.ds(base, CHUNK)], ssem
        ).wait()


def kernel(inputs, embedding):
    idx = inputs.astype(jnp.int32).reshape(NW, NCHUNK, CHUNK)
    table = jnp.asarray(embedding, jnp.float32)
    out = _embed_gather(idx, table)
    return out.reshape(BATCH, FIELDS, FEATURES)


# speedup vs baseline: 1.0567x; 1.0407x over previous
import functools

import jax
import jax.numpy as jnp
from jax import lax
from jax.experimental import pallas as pl
from jax.experimental.pallas import tpu as pltpu
from jax.experimental.pallas import tpu_sc as plsc

NUM_EMBEDDINGS = 1000000
FEATURES = 64
BATCH = 16384
FIELDS = 26

NC = 2
NS = 16
NW = NC * NS

B_TOTAL = BATCH * FIELDS
BPW = B_TOTAL // NW
CHUNK = 128
NCHUNK = BPW // CHUNK
K = 4
NGROUP = NCHUNK // K
NBUF = 3 * K

_mesh = plsc.VectorSubcoreMesh(core_axis_name="c", subcore_axis_name="s")


@functools.partial(
    pl.kernel,
    mesh=_mesh,
    out_type=jax.ShapeDtypeStruct((B_TOTAL, FEATURES), jnp.float32),
    scratch_types=[
        pltpu.VMEM((BPW,), jnp.int32),
        pltpu.VMEM((NBUF, CHUNK, FEATURES), jnp.float32),
        pltpu.SemaphoreType.DMA,
        pltpu.SemaphoreType.DMA,
    ],
    compiler_params=pltpu.CompilerParams(use_tc_tiling_on_sc=False),
)
def _embed_gather(idx_hbm, table_hbm, out_hbm, idx_v, rows_v, gsem, ssem):
    wid = lax.axis_index("s") * NC + lax.axis_index("c")
    base = wid * BPW

    pltpu.sync_copy(idx_hbm.at[pl.ds(base, BPW)], idx_v)

    def idx_ref(c):
        return idx_v.at[pl.ds(c * CHUNK, CHUNK)]

    for j in range(K):
        pltpu.async_copy(table_hbm.at[idx_ref(j)], rows_v.at[j], gsem)

    def body(i, carry):
        @pl.when(i >= 2)
        def _():
            for j in range(K):
                pltpu.make_async_copy(
                    rows_v.at[j], out_hbm.at[pl.ds(base, CHUNK)], ssem
                ).wait()

        @pl.when(i + 1 < NGROUP)
        def _():
            boff = lax.rem(i + 1, 3) * K
            for j in range(K):
                pltpu.async_copy(
                    table_hbm.at[idx_ref((i + 1) * K + j)],
                    rows_v.at[boff + j],
                    gsem,
                )

        boff = lax.rem(i, 3) * K
        for j in range(K):
            c = i * K + j
            pltpu.make_async_copy(
                table_hbm.at[idx_ref(c)], rows_v.at[boff + j], gsem
            ).wait()
            pltpu.async_copy(
                rows_v.at[boff + j],
                out_hbm.at[pl.ds(base + c * CHUNK, CHUNK)],
                ssem,
            )
        return carry

    lax.fori_loop(0, NGROUP, body, 0)

    for j in range(2 * K):
        pltpu.make_async_copy(
            rows_v.at[0], out_hbm.at[pl.ds(base, CHUNK)], ssem
        ).wait()


def kernel(inputs, embedding):
    idx_flat = inputs.astype(jnp.int32).T.reshape(-1)
    table = jnp.asarray(embedding, jnp.float32)
    out = _embed_gather(idx_flat, table)
    return out.reshape(FIELDS, BATCH, FEATURES).transpose(1, 0, 2)
